# baseline (speedup 1.0000x reference)
import jax
import jax.numpy as jnp
from jax.experimental import pallas as pl
from jax.experimental.pallas import tpu as pltpu

B, HORIZON, NUM_COV, HID = 8192, 24, 7, 512
INP = NUM_COV + 1
G4 = 4 * HID
BB = 512
NB = B // BB


def _decoder_kernel(x_ref, h0_ref, c0_ref, y0_ref, wx_ref, wy_ref, whh_ref,
                    b_ref, fcw_ref, fcb_ref, out_ref):
    h = h0_ref[...]
    c = c0_ref[...]
    y = y0_ref[...]
    wx = wx_ref[...].astype(jnp.bfloat16)
    wy = wy_ref[...]
    whh = whh_ref[...].astype(jnp.bfloat16)
    b = b_ref[...]
    fcw = fcw_ref[...]
    fcb = fcb_ref[0, 0]

    for t in range(HORIZON):
        x_t = x_ref[:, t, :].astype(jnp.bfloat16)
        gates = (
            jnp.dot(h.astype(jnp.bfloat16), whh,
                    preferred_element_type=jnp.float32)
            + jnp.dot(x_t, wx, preferred_element_type=jnp.float32)
            + y * wy
            + b
        )
        i = jax.nn.sigmoid(gates[:, 0 * HID:1 * HID])
        f = jax.nn.sigmoid(gates[:, 1 * HID:2 * HID])
        g = jnp.tanh(gates[:, 2 * HID:3 * HID])
        o = jax.nn.sigmoid(gates[:, 3 * HID:4 * HID])
        c = f * c + i * g
        h = o * jnp.tanh(c)
        logit = jnp.sum(h * fcw, axis=1, keepdims=True) + fcb
        y = jax.nn.sigmoid(logit)
        out_ref[:, t:t + 1] = logit


def kernel(future_x, h_enc, c_enc, y0, W_ih, W_hh, b_ih, b_hh, fc_w, fc_b):
    wx = W_ih[:, :NUM_COV].T
    wy = W_ih[:, NUM_COV:].T
    whh = W_hh.T
    b = (b_ih + b_hh).reshape(1, G4)
    fcb = fc_b.reshape(1, 1)

    out = pl.pallas_call(
        _decoder_kernel,
        grid=(NB,),
        in_specs=[
            pl.BlockSpec((BB, HORIZON, NUM_COV), lambda i: (i, 0, 0)),
            pl.BlockSpec((BB, HID), lambda i: (i, 0)),
            pl.BlockSpec((BB, HID), lambda i: (i, 0)),
            pl.BlockSpec((BB, 1), lambda i: (i, 0)),
            pl.BlockSpec((NUM_COV, G4), lambda i: (0, 0)),
            pl.BlockSpec((1, G4), lambda i: (0, 0)),
            pl.BlockSpec((HID, G4), lambda i: (0, 0)),
            pl.BlockSpec((1, G4), lambda i: (0, 0)),
            pl.BlockSpec((1, HID), lambda i: (0, 0)),
            pl.BlockSpec((1, 1), lambda i: (0, 0)),
        ],
        out_specs=pl.BlockSpec((BB, HORIZON), lambda i: (i, 0)),
        out_shape=jax.ShapeDtypeStruct((B, HORIZON), jnp.float32),
        compiler_params=pltpu.CompilerParams(
            dimension_semantics=("parallel",),
            vmem_limit_bytes=56 * 1024 * 1024,
        ),
    )(future_x, h_enc, c_enc, y0, wx, wy, whh, b, fc_w, fcb)
    return out[..., None]


# speedup vs baseline: 1.2147x; 1.2147x over previous
import jax
import jax.numpy as jnp
from jax.experimental import pallas as pl
from jax.experimental.pallas import tpu as pltpu

B, HORIZON, NUM_COV, HID = 8192, 24, 7, 512
INP = NUM_COV + 1
G4 = 4 * HID
KA = NUM_COV + 2
BB = 1024
NB = B // BB


def _decoder_kernel(x_ref, h0_ref, c0_ref, y0_ref, wxa_ref, whh_ref,
                    fcw_ref, fcb_ref, out_ref):
    h = h0_ref[...]
    c = c0_ref[...]
    y = y0_ref[...]
    wxa = wxa_ref[...]
    whh = whh_ref[...]
    fcw = fcw_ref[...]
    fcb = fcb_ref[0, 0]
    ones_col = jnp.ones((BB, 1), jnp.float32)

    for t in range(HORIZON):
        x_aug = jnp.concatenate([x_ref[:, t, :], y, ones_col], axis=1)
        gates = (
            jnp.dot(h, whh, preferred_element_type=jnp.float32)
            + jnp.dot(x_aug, wxa, preferred_element_type=jnp.float32)
        )
        i = jax.nn.sigmoid(gates[:, 0 * HID:1 * HID])
        f = jax.nn.sigmoid(gates[:, 1 * HID:2 * HID])
        g = jnp.tanh(gates[:, 2 * HID:3 * HID])
        o = jax.nn.sigmoid(gates[:, 3 * HID:4 * HID])
        c = f * c + i * g
        h = o * jnp.tanh(c)
        logit = jnp.sum(h * fcw, axis=1, keepdims=True) + fcb
        y = jax.nn.sigmoid(logit)
        out_ref[:, t:t + 1] = logit


def kernel(future_x, h_enc, c_enc, y0, W_ih, W_hh, b_ih, b_hh, fc_w, fc_b):
    wxa = jnp.concatenate(
        [W_ih.T, (b_ih + b_hh).reshape(1, G4)], axis=0)
    whh = W_hh.T
    fcb = fc_b.reshape(1, 1)

    out = pl.pallas_call(
        _decoder_kernel,
        grid=(NB,),
        in_specs=[
            pl.BlockSpec((BB, HORIZON, NUM_COV), lambda i: (i, 0, 0)),
            pl.BlockSpec((BB, HID), lambda i: (i, 0)),
            pl.BlockSpec((BB, HID), lambda i: (i, 0)),
            pl.BlockSpec((BB, 1), lambda i: (i, 0)),
            pl.BlockSpec((KA, G4), lambda i: (0, 0)),
            pl.BlockSpec((HID, G4), lambda i: (0, 0)),
            pl.BlockSpec((1, HID), lambda i: (0, 0)),
            pl.BlockSpec((1, 1), lambda i: (0, 0)),
        ],
        out_specs=pl.BlockSpec((BB, HORIZON), lambda i: (i, 0)),
        out_shape=jax.ShapeDtypeStruct((B, HORIZON), jnp.float32),
        compiler_params=pltpu.CompilerParams(
            dimension_semantics=("parallel",),
            vmem_limit_bytes=56 * 1024 * 1024,
        ),
    )(future_x, h_enc, c_enc, y0, wxa, whh, fc_w, fcb)
    return out[..., None]


# speedup vs baseline: 1.2980x; 1.0685x over previous
import jax
import jax.numpy as jnp
from jax.experimental import pallas as pl
from jax.experimental.pallas import tpu as pltpu

B, HORIZON, NUM_COV, HID = 8192, 24, 7, 512
INP = NUM_COV + 1
G4 = 4 * HID
KA = NUM_COV + 2
BB = 1024
NB = B // BB


NCHAIN = 2
CB = BB // NCHAIN


def _decoder_kernel(x_ref, h0_ref, c0_ref, y0_ref, wxa_ref, whh_ref,
                    fcw_ref, fcb_ref, out_ref):
    wxa = wxa_ref[...]
    whh = whh_ref[...]
    fcw = fcw_ref[...]
    fcb = fcb_ref[0, 0]
    ones_col = jnp.ones((CB, 1), jnp.float32)

    hs = [h0_ref[q * CB:(q + 1) * CB, :] for q in range(NCHAIN)]
    cs = [c0_ref[q * CB:(q + 1) * CB, :] for q in range(NCHAIN)]
    ys = [y0_ref[q * CB:(q + 1) * CB, :] for q in range(NCHAIN)]

    for t in range(HORIZON):
        for q in range(NCHAIN):
            lo = q * CB
            x_aug = jnp.concatenate(
                [x_ref[lo:lo + CB, t, :], ys[q], ones_col], axis=1)
            gates = (
                jnp.dot(hs[q], whh, preferred_element_type=jnp.float32)
                + jnp.dot(x_aug, wxa, preferred_element_type=jnp.float32)
            )
            i = jax.nn.sigmoid(gates[:, 0 * HID:1 * HID])
            f = jax.nn.sigmoid(gates[:, 1 * HID:2 * HID])
            g = jnp.tanh(gates[:, 2 * HID:3 * HID])
            o = jax.nn.sigmoid(gates[:, 3 * HID:4 * HID])
            cs[q] = f * cs[q] + i * g
            hs[q] = o * jnp.tanh(cs[q])
            logit = jnp.sum(hs[q] * fcw, axis=1, keepdims=True) + fcb
            ys[q] = jax.nn.sigmoid(logit)
            out_ref[lo:lo + CB, t:t + 1] = logit


def kernel(future_x, h_enc, c_enc, y0, W_ih, W_hh, b_ih, b_hh, fc_w, fc_b):
    wxa = jnp.concatenate(
        [W_ih.T, (b_ih + b_hh).reshape(1, G4)], axis=0)
    whh = W_hh.T
    fcb = fc_b.reshape(1, 1)

    out = pl.pallas_call(
        _decoder_kernel,
        grid=(NB,),
        in_specs=[
            pl.BlockSpec((BB, HORIZON, NUM_COV), lambda i: (i, 0, 0)),
            pl.BlockSpec((BB, HID), lambda i: (i, 0)),
            pl.BlockSpec((BB, HID), lambda i: (i, 0)),
            pl.BlockSpec((BB, 1), lambda i: (i, 0)),
            pl.BlockSpec((KA, G4), lambda i: (0, 0)),
            pl.BlockSpec((HID, G4), lambda i: (0, 0)),
            pl.BlockSpec((1, HID), lambda i: (0, 0)),
            pl.BlockSpec((1, 1), lambda i: (0, 0)),
        ],
        out_specs=pl.BlockSpec((BB, HORIZON), lambda i: (i, 0)),
        out_shape=jax.ShapeDtypeStruct((B, HORIZON), jnp.float32),
        compiler_params=pltpu.CompilerParams(
            dimension_semantics=("parallel",),
            vmem_limit_bytes=56 * 1024 * 1024,
        ),
    )(future_x, h_enc, c_enc, y0, wxa, whh, fc_w, fcb)
    return out[..., None]


# speedup vs baseline: 1.3119x; 1.0107x over previous
import jax
import jax.numpy as jnp
from jax.experimental import pallas as pl
from jax.experimental.pallas import tpu as pltpu

B, HORIZON, NUM_COV, HID = 8192, 24, 7, 512
INP = NUM_COV + 1
G4 = 4 * HID
KA = NUM_COV + 2
BB = 1024
NB = B // BB


NCHAIN = 2
CB = BB // NCHAIN


def _sigmoid(x):
    return 0.5 * jnp.tanh(0.5 * x) + 0.5


def _decoder_kernel(x_ref, h0_ref, c0_ref, y0_ref, wxa_ref, whh_ref,
                    fcw_ref, fcb_ref, out_ref):
    wxa = wxa_ref[...]
    whh = whh_ref[...]
    fcw = fcw_ref[...]
    fcb = fcb_ref[0, 0]
    ones_col = jnp.ones((CB, 1), jnp.float32)

    hs = [h0_ref[q * CB:(q + 1) * CB, :] for q in range(NCHAIN)]
    cs = [c0_ref[q * CB:(q + 1) * CB, :] for q in range(NCHAIN)]
    ys = [y0_ref[q * CB:(q + 1) * CB, :] for q in range(NCHAIN)]

    for t in range(HORIZON):
        for q in range(NCHAIN):
            lo = q * CB
            x_aug = jnp.concatenate(
                [x_ref[lo:lo + CB, t, :], ys[q], ones_col], axis=1)
            gates = (
                jnp.dot(hs[q], whh, preferred_element_type=jnp.float32)
                + jnp.dot(x_aug, wxa, preferred_element_type=jnp.float32)
            )
            i = _sigmoid(gates[:, 0 * HID:1 * HID])
            f = _sigmoid(gates[:, 1 * HID:2 * HID])
            g = jnp.tanh(gates[:, 2 * HID:3 * HID])
            o = _sigmoid(gates[:, 3 * HID:4 * HID])
            cs[q] = f * cs[q] + i * g
            hs[q] = o * jnp.tanh(cs[q])
            logit = jnp.sum(hs[q] * fcw, axis=1, keepdims=True) + fcb
            ys[q] = _sigmoid(logit)
            out_ref[lo:lo + CB, t:t + 1] = logit


def kernel(future_x, h_enc, c_enc, y0, W_ih, W_hh, b_ih, b_hh, fc_w, fc_b):
    wxa = jnp.concatenate(
        [W_ih.T, (b_ih + b_hh).reshape(1, G4)], axis=0)
    whh = W_hh.T
    fcb = fc_b.reshape(1, 1)

    out = pl.pallas_call(
        _decoder_kernel,
        grid=(NB,),
        in_specs=[
            pl.BlockSpec((BB, HORIZON, NUM_COV), lambda i: (i, 0, 0)),
            pl.BlockSpec((BB, HID), lambda i: (i, 0)),
            pl.BlockSpec((BB, HID), lambda i: (i, 0)),
            pl.BlockSpec((BB, 1), lambda i: (i, 0)),
            pl.BlockSpec((KA, G4), lambda i: (0, 0)),
            pl.BlockSpec((HID, G4), lambda i: (0, 0)),
            pl.BlockSpec((1, HID), lambda i: (0, 0)),
            pl.BlockSpec((1, 1), lambda i: (0, 0)),
        ],
        out_specs=pl.BlockSpec((BB, HORIZON), lambda i: (i, 0)),
        out_shape=jax.ShapeDtypeStruct((B, HORIZON), jnp.float32),
        compiler_params=pltpu.CompilerParams(
            dimension_semantics=("parallel",),
            vmem_limit_bytes=56 * 1024 * 1024,
        ),
    )(future_x, h_enc, c_enc, y0, wxa, whh, fc_w, fcb)
    return out[..., None]
